# Initial kernel scaffold; baseline (speedup 1.0000x reference)
import jax
import jax.numpy as jnp
from jax.experimental import pallas as pl


def kernel(zt, pred, Qt, Qsb, Qtb, batch):
    raise NotImplementedError("write your pallas kernel here")


# speedup vs baseline: 16.0703x; 16.0703x over previous
import jax
import jax.numpy as jnp
from jax import lax
from jax.experimental import pallas as pl
from jax.experimental.pallas import tpu as pltpu


def _block_kernel(blo_ref, bhi_ref, batch_ref, zt_ref, pred_ref,
                  qtT_ref, qsb_ref, qtbT_ref, out_ref):
    i = pl.program_id(0)
    b_lo = blo_ref[i]
    b_hi = bhi_ref[i]
    zt = zt_ref[...]
    pred = pred_ref[...]
    bidx = batch_ref[...]

    m = jnp.max(pred, axis=-1, keepdims=True)
    e = jnp.exp(pred - m)
    pred_x = e / jnp.sum(e, axis=-1, keepdims=True)

    r, c = zt.shape

    def body(b, acc):
        qtT = qtT_ref[b]
        qsb = qsb_ref[b]
        qtbT = qtbT_ref[b]
        left = jnp.dot(zt, qtT, preferred_element_type=jnp.float32)
        den = jnp.dot(zt, qtbT, preferred_element_type=jnp.float32)
        den = jnp.where(den == 0.0, 1e-6, den)
        w = pred_x / den
        s = jnp.dot(w, qsb, preferred_element_type=jnp.float32)
        contrib = left * s
        mask = bidx == b
        return acc + jnp.where(mask, contrib, 0.0)

    un = lax.fori_loop(b_lo, b_hi + 1, body,
                       jnp.zeros((r, c), jnp.float32))
    un = jnp.where(un <= 0.0, 1e-5, un)
    out_ref[...] = un / jnp.sum(un, axis=-1, keepdims=True)


def kernel(zt, pred, Qt, Qsb, Qtb, batch):
    n, c = zt.shape
    bs = Qt.shape[0]
    r = 512
    nb = n // r

    batch = batch.astype(jnp.int32)
    qtT = jnp.swapaxes(Qt, 1, 2)
    qtbT = jnp.swapaxes(Qtb, 1, 2)
    bmat = batch.reshape(nb, r)
    blo = bmat[:, 0]
    bhi = bmat[:, -1]
    batch2d = batch.reshape(n, 1)

    grid_spec = pltpu.PrefetchScalarGridSpec(
        num_scalar_prefetch=2,
        grid=(nb,),
        in_specs=[
            pl.BlockSpec((r, 1), lambda i, *_: (i, 0)),
            pl.BlockSpec((r, c), lambda i, *_: (i, 0)),
            pl.BlockSpec((r, c), lambda i, *_: (i, 0)),
            pl.BlockSpec((bs, c, c), lambda i, *_: (0, 0, 0)),
            pl.BlockSpec((bs, c, c), lambda i, *_: (0, 0, 0)),
            pl.BlockSpec((bs, c, c), lambda i, *_: (0, 0, 0)),
        ],
        out_specs=pl.BlockSpec((r, c), lambda i, *_: (i, 0)),
    )
    return pl.pallas_call(
        _block_kernel,
        grid_spec=grid_spec,
        out_shape=jax.ShapeDtypeStruct((n, c), jnp.float32),
    )(blo, bhi, batch2d, zt, pred, qtT, Qsb, qtbT)


# speedup vs baseline: 21.2573x; 1.3228x over previous
import jax
import jax.numpy as jnp
from jax import lax
from jax.experimental import pallas as pl
from jax.experimental.pallas import tpu as pltpu


_UNROLL = 3


def _block_kernel(blo_ref, bhi_ref, batch_ref, zt_ref, pred_ref,
                  qcat_ref, qsb_ref, out_ref):
    i = pl.program_id(0)
    b_lo = blo_ref[i]
    b_hi = bhi_ref[i]
    zt = zt_ref[...]
    pred = pred_ref[...]
    bidx = batch_ref[...]

    m = jnp.max(pred, axis=-1, keepdims=True)
    e = jnp.exp(pred - m)
    pred_x = e / jnp.sum(e, axis=-1, keepdims=True)

    r, c = zt.shape
    bs = qsb_ref.shape[0]

    def body(b, acc):
        bb = jnp.minimum(b, bs - 1)
        ld = jnp.dot(zt, qcat_ref[bb], preferred_element_type=jnp.float32)
        left = ld[:, :c]
        den = ld[:, c:]
        den = jnp.where(den == 0.0, 1e-6, den)
        w = pred_x / den
        s = jnp.dot(w, qsb_ref[bb], preferred_element_type=jnp.float32)
        mask = bidx == b
        return acc + jnp.where(mask, left * s, 0.0)

    acc = jnp.zeros((r, c), jnp.float32)
    for j in range(_UNROLL):
        acc = body(b_lo + j, acc)
    un = lax.fori_loop(b_lo + _UNROLL, b_hi + 1, body, acc)
    un = jnp.where(un <= 0.0, 1e-5, un)
    out_ref[...] = un / jnp.sum(un, axis=-1, keepdims=True)


def kernel(zt, pred, Qt, Qsb, Qtb, batch):
    n, c = zt.shape
    bs = Qt.shape[0]
    r = 1024
    nb = n // r

    batch = batch.astype(jnp.int32)
    qcat = jnp.concatenate(
        [jnp.swapaxes(Qt, 1, 2), jnp.swapaxes(Qtb, 1, 2)], axis=2)
    bmat = batch.reshape(nb, r)
    blo = bmat[:, 0]
    bhi = bmat[:, -1]
    batch2d = batch.reshape(n, 1)

    grid_spec = pltpu.PrefetchScalarGridSpec(
        num_scalar_prefetch=2,
        grid=(nb,),
        in_specs=[
            pl.BlockSpec((r, 1), lambda i, *_: (i, 0)),
            pl.BlockSpec((r, c), lambda i, *_: (i, 0)),
            pl.BlockSpec((r, c), lambda i, *_: (i, 0)),
            pl.BlockSpec((bs, c, 2 * c), lambda i, *_: (0, 0, 0)),
            pl.BlockSpec((bs, c, c), lambda i, *_: (0, 0, 0)),
        ],
        out_specs=pl.BlockSpec((r, c), lambda i, *_: (i, 0)),
    )
    return pl.pallas_call(
        _block_kernel,
        grid_spec=grid_spec,
        out_shape=jax.ShapeDtypeStruct((n, c), jnp.float32),
    )(blo, bhi, batch2d, zt, pred, qcat, Qsb)
